# Initial kernel scaffold; baseline (speedup 1.0000x reference)
import jax
import jax.numpy as jnp
from jax.experimental import pallas as pl


def kernel(x, edge_index, W1, b1, W2, b2):
    raise NotImplementedError("write your pallas kernel here")


# speedup vs baseline: 10.4439x; 10.4439x over previous
import functools

import jax
import jax.numpy as jnp
from jax import lax
from jax.experimental import pallas as pl
from jax.experimental.pallas import tpu as pltpu
from jax.experimental.pallas import tpu_sc as plsc

N = 10000
E = 320000
D_IN = 128
D_H = 128
D_OUT = 64

NC = 2
NS = 16
NW = NC * NS
CH = 128
K = -(-E // (NW * CH))
E_PAD = NW * CH * K
N_ACC = ((N + 1) + 1023) // 1024 * 1024
RPT = N_ACC // NS

_mesh = plsc.VectorSubcoreMesh(core_axis_name="c", subcore_axis_name="s")


def _make_msg_kernel(d):
    @functools.partial(
        pl.kernel,
        out_type=jax.ShapeDtypeStruct((NC, N_ACC, d), jnp.float32),
        mesh=_mesh,
        scratch_types=[
            pltpu.VMEM((CH,), jnp.int32),
            pltpu.VMEM((CH,), jnp.int32),
            pltpu.VMEM((CH, d), jnp.float32),
            pltpu.VMEM_SHARED((N_ACC, d), jnp.float32),
            pltpu.SemaphoreType.DMA,
        ],
    )
    def msg(g_hbm, src_hbm, dst_hbm, zero_hbm, out_hbm, sidx, didx, rows, acc, sem):
        c = lax.axis_index("c")
        s = lax.axis_index("s")
        wid = s * NC + c
        r0 = s * RPT
        pltpu.sync_copy(zero_hbm.at[pl.ds(r0, RPT)], acc.at[pl.ds(r0, RPT)])
        plsc.subcore_barrier()
        base = wid * (K * CH)

        def body(j, carry):
            off = pl.multiple_of(base + j * CH, CH)
            pltpu.sync_copy(src_hbm.at[pl.ds(off, CH)], sidx)
            pltpu.sync_copy(dst_hbm.at[pl.ds(off, CH)], didx)
            pltpu.async_copy(g_hbm.at[sidx], rows, sem).wait()
            pltpu.sync_copy(rows, acc.at[didx], add=True)
            return carry

        lax.fori_loop(0, K, body, 0)
        plsc.subcore_barrier()
        pltpu.sync_copy(acc.at[pl.ds(r0, RPT)], out_hbm.at[c, pl.ds(r0, RPT)])

    return msg


@functools.partial(
    pl.kernel,
    out_type=jax.ShapeDtypeStruct((NC, N_ACC, D_H), jnp.float32),
    mesh=_mesh,
    scratch_types=[
        pltpu.VMEM((CH,), jnp.int32),
        pltpu.VMEM((CH, D_H), jnp.float32),
        pltpu.VMEM_SHARED((N_ACC, D_H), jnp.float32),
    ],
)
def _deg_kernel(dst_hbm, ones_hbm, zero_hbm, out_hbm, didx, ones_v, acc):
    c = lax.axis_index("c")
    s = lax.axis_index("s")
    wid = s * NC + c
    r0 = s * RPT
    pltpu.sync_copy(zero_hbm.at[pl.ds(r0, RPT)], acc.at[pl.ds(r0, RPT)])
    pltpu.sync_copy(ones_hbm, ones_v)
    plsc.subcore_barrier()
    base = wid * (K * CH)

    def body(j, carry):
        off = pl.multiple_of(base + j * CH, CH)
        pltpu.sync_copy(dst_hbm.at[pl.ds(off, CH)], didx)
        pltpu.sync_copy(ones_v, acc.at[didx], add=True)
        return carry

    lax.fori_loop(0, K, body, 0)
    plsc.subcore_barrier()
    pltpu.sync_copy(acc.at[pl.ds(r0, RPT)], out_hbm.at[c, pl.ds(r0, RPT)])


BR = 1000


def _dinv_of(degp_ref):
    deg = degp_ref[0, :, 0] + degp_ref[1, :, 0] + 1.0
    return lax.rsqrt(jnp.maximum(deg, 1.0))


def _tc1_body(degp_ref, x_ref, w_ref, g_ref):
    dinv = _dinv_of(degp_ref)
    h = jnp.dot(x_ref[...], w_ref[...], preferred_element_type=jnp.float32)
    g_ref[...] = h * dinv[:, None]


def _tc2_body(degp_ref, p_ref, g1_ref, b1_ref, w2_ref, g2_ref):
    dinv = _dinv_of(degp_ref)
    ssum = p_ref[0] + p_ref[1] + g1_ref[...]
    h = jnp.maximum(ssum * dinv[:, None] + b1_ref[0], 0.0)
    h2 = jnp.dot(h, w2_ref[...], preferred_element_type=jnp.float32)
    g2_ref[...] = jnp.pad(h2 * dinv[:, None], ((0, 0), (0, D_H - D_OUT)))


def _tc3_body(degp_ref, p_ref, g2_ref, b2_ref, o_ref):
    dinv = _dinv_of(degp_ref)
    ssum = p_ref[0, :, :D_OUT] + p_ref[1, :, :D_OUT] + g2_ref[:, :D_OUT]
    z = ssum * dinv[:, None] + b2_ref[0]
    m = jnp.max(z, axis=1, keepdims=True)
    zz = z - m
    lse = jnp.log(jnp.sum(jnp.exp(zz), axis=1, keepdims=True))
    o_ref[...] = zz - lse


def _degp_spec():
    return pl.BlockSpec((NC, BR, D_H), lambda i: (0, i, 0))


def kernel(x, edge_index, W1, b1, W2, b2):
    src = edge_index[0]
    dst = edge_index[1]
    pad = E_PAD - E
    if pad:
        src = jnp.concatenate([src, jnp.zeros((pad,), jnp.int32)])
        dst = jnp.concatenate([dst, jnp.full((pad,), N, jnp.int32)])

    zeros128 = jnp.zeros((N_ACC, D_H), jnp.float32)
    ones128 = jnp.ones((CH, D_H), jnp.float32)

    degp = _deg_kernel(dst, ones128, zeros128)

    g1 = pl.pallas_call(
        _tc1_body,
        grid=(N // BR,),
        in_specs=[
            _degp_spec(),
            pl.BlockSpec((BR, D_IN), lambda i: (i, 0)),
            pl.BlockSpec((D_IN, D_H), lambda i: (0, 0)),
        ],
        out_specs=pl.BlockSpec((BR, D_H), lambda i: (i, 0)),
        out_shape=jax.ShapeDtypeStruct((N, D_H), jnp.float32),
    )(degp, x, W1)

    p1 = _make_msg_kernel(D_H)(g1, src, dst, zeros128)

    g2 = pl.pallas_call(
        _tc2_body,
        grid=(N // BR,),
        in_specs=[
            _degp_spec(),
            pl.BlockSpec((NC, BR, D_H), lambda i: (0, i, 0)),
            pl.BlockSpec((BR, D_H), lambda i: (i, 0)),
            pl.BlockSpec((1, D_H), lambda i: (0, 0)),
            pl.BlockSpec((D_H, D_OUT), lambda i: (0, 0)),
        ],
        out_specs=pl.BlockSpec((BR, D_H), lambda i: (i, 0)),
        out_shape=jax.ShapeDtypeStruct((N, D_H), jnp.float32),
    )(degp, p1, g1, b1.reshape(1, D_H), W2)

    p2 = _make_msg_kernel(D_H)(g2, src, dst, zeros128)

    out = pl.pallas_call(
        _tc3_body,
        grid=(N // BR,),
        in_specs=[
            _degp_spec(),
            pl.BlockSpec((NC, BR, D_H), lambda i: (0, i, 0)),
            pl.BlockSpec((BR, D_H), lambda i: (i, 0)),
            pl.BlockSpec((1, D_OUT), lambda i: (0, 0)),
        ],
        out_specs=pl.BlockSpec((BR, D_OUT), lambda i: (i, 0)),
        out_shape=jax.ShapeDtypeStruct((N, D_OUT), jnp.float32),
    )(degp, p2, g2, b2.reshape(1, D_OUT))

    return out


# speedup vs baseline: 16.1211x; 1.5436x over previous
import functools

import jax
import jax.numpy as jnp
from jax import lax
from jax.experimental import pallas as pl
from jax.experimental.pallas import tpu as pltpu
from jax.experimental.pallas import tpu_sc as plsc

N = 10000
E = 320000
D_IN = 128
D_H = 128
D_OUT = 64

NC = 2
NS = 16
NW = NC * NS
CH = 112
K = 90
E_PAD = NW * CH * K
N_ACC = ((N + 1) + 15) // 16 * 16
RPT = N_ACC // NS

_mesh = plsc.VectorSubcoreMesh(core_axis_name="c", subcore_axis_name="s")


def _make_msg_kernel(d):
    @functools.partial(
        pl.kernel,
        out_type=jax.ShapeDtypeStruct((NC, N_ACC, d), jnp.float32),
        mesh=_mesh,
        scratch_types=[
            pltpu.VMEM((K, CH), jnp.int32),
            pltpu.VMEM((K, CH), jnp.int32),
            pltpu.VMEM((CH, d), jnp.float32),
            pltpu.VMEM((CH, d), jnp.float32),
            pltpu.VMEM_SHARED((N_ACC, d), jnp.float32),
            pltpu.SemaphoreType.DMA,
            pltpu.SemaphoreType.DMA,
        ],
        compiler_params=pltpu.CompilerParams(use_tc_tiling_on_sc=False),
    )
    def msg(g_hbm, src_hbm, dst_hbm, zero_hbm, out_hbm,
            sidx, didx, rows_a, rows_b, acc, sem_a, sem_b):
        c = lax.axis_index("c")
        s = lax.axis_index("s")
        wid = s * NC + c
        r0 = s * RPT
        pltpu.sync_copy(zero_hbm.at[pl.ds(r0, RPT)], acc.at[pl.ds(r0, RPT)])
        pltpu.sync_copy(src_hbm.at[wid], sidx)
        pltpu.sync_copy(dst_hbm.at[wid], didx)
        plsc.subcore_barrier()

        pltpu.async_copy(g_hbm.at[sidx.at[0]], rows_a, sem_a)

        def body(jj, carry):
            j0 = 2 * jj
            pltpu.make_async_copy(g_hbm.at[sidx.at[j0]], rows_a, sem_a).wait()
            pltpu.async_copy(g_hbm.at[sidx.at[j0 + 1]], rows_b, sem_b)
            pltpu.sync_copy(rows_a, acc.at[didx.at[j0]], add=True)
            jn = jnp.minimum(j0 + 2, K - 1)
            pltpu.make_async_copy(g_hbm.at[sidx.at[j0 + 1]], rows_b, sem_b).wait()
            pltpu.async_copy(g_hbm.at[sidx.at[jn]], rows_a, sem_a)
            pltpu.sync_copy(rows_b, acc.at[didx.at[j0 + 1]], add=True)
            return carry

        lax.fori_loop(0, K // 2, body, 0)
        pltpu.make_async_copy(g_hbm.at[sidx.at[K - 1]], rows_a, sem_a).wait()
        plsc.subcore_barrier()
        pltpu.sync_copy(acc.at[pl.ds(r0, RPT)], out_hbm.at[c, pl.ds(r0, RPT)])

    return msg


DEG_G = 9


@functools.partial(
    pl.kernel,
    out_type=jax.ShapeDtypeStruct((NC, N_ACC, D_H), jnp.float32),
    mesh=_mesh,
    scratch_types=[
        pltpu.VMEM((K, CH), jnp.int32),
        pltpu.VMEM((CH, D_H), jnp.float32),
        pltpu.VMEM_SHARED((N_ACC, D_H), jnp.float32),
        pltpu.SemaphoreType.DMA,
    ],
    compiler_params=pltpu.CompilerParams(use_tc_tiling_on_sc=False),
)
def _deg_kernel(dst_hbm, ones_hbm, zero_hbm, out_hbm, didx, ones_v, acc, sem):
    c = lax.axis_index("c")
    s = lax.axis_index("s")
    wid = s * NC + c
    r0 = s * RPT
    pltpu.sync_copy(zero_hbm.at[pl.ds(r0, RPT)], acc.at[pl.ds(r0, RPT)])
    pltpu.sync_copy(dst_hbm.at[wid], didx)
    pltpu.sync_copy(ones_hbm, ones_v)
    plsc.subcore_barrier()

    def body(g, carry):
        for b in range(DEG_G):
            pltpu.async_copy(ones_v, acc.at[didx.at[g * DEG_G + b]], sem, add=True)
        for b in range(DEG_G):
            pltpu.make_async_copy(ones_v, acc.at[didx.at[g * DEG_G + b]], sem).wait()
        return carry

    lax.fori_loop(0, K // DEG_G, body, 0)
    plsc.subcore_barrier()
    pltpu.sync_copy(acc.at[pl.ds(r0, RPT)], out_hbm.at[c, pl.ds(r0, RPT)])


BR = 1000


def _dinv_of(degp_ref):
    deg = degp_ref[0, :, 0] + degp_ref[1, :, 0] + 1.0
    return lax.rsqrt(jnp.maximum(deg, 1.0))


def _tc1_body(degp_ref, x_ref, w_ref, g_ref):
    dinv = _dinv_of(degp_ref)
    h = jnp.dot(x_ref[...], w_ref[...], preferred_element_type=jnp.float32)
    g_ref[...] = h * dinv[:, None]


def _tc2_body(degp_ref, p_ref, g1_ref, b1_ref, w2_ref, g2_ref):
    dinv = _dinv_of(degp_ref)
    ssum = p_ref[0] + p_ref[1] + g1_ref[...]
    h = jnp.maximum(ssum * dinv[:, None] + b1_ref[0], 0.0)
    h2 = jnp.dot(h, w2_ref[...], preferred_element_type=jnp.float32)
    g2_ref[...] = jnp.pad(h2 * dinv[:, None], ((0, 0), (0, D_H - D_OUT)))


def _tc3_body(degp_ref, p_ref, g2_ref, b2_ref, o_ref):
    dinv = _dinv_of(degp_ref)
    ssum = p_ref[0, :, :D_OUT] + p_ref[1, :, :D_OUT] + g2_ref[:, :D_OUT]
    z = ssum * dinv[:, None] + b2_ref[0]
    m = jnp.max(z, axis=1, keepdims=True)
    zz = z - m
    lse = jnp.log(jnp.sum(jnp.exp(zz), axis=1, keepdims=True))
    o_ref[...] = zz - lse


def _degp_spec():
    return pl.BlockSpec((NC, BR, D_H), lambda i: (0, i, 0))


def kernel(x, edge_index, W1, b1, W2, b2):
    src = edge_index[0]
    dst = edge_index[1]
    pad = E_PAD - E
    src = jnp.concatenate([src, jnp.zeros((pad,), jnp.int32)]).reshape(NW, K, CH)
    dst = jnp.concatenate([dst, jnp.full((pad,), N, jnp.int32)]).reshape(NW, K, CH)

    zeros128 = jnp.zeros((N_ACC, D_H), jnp.float32)
    ones128 = jnp.ones((CH, D_H), jnp.float32)

    degp = _deg_kernel(dst, ones128, zeros128)

    g1 = pl.pallas_call(
        _tc1_body,
        grid=(N // BR,),
        in_specs=[
            _degp_spec(),
            pl.BlockSpec((BR, D_IN), lambda i: (i, 0)),
            pl.BlockSpec((D_IN, D_H), lambda i: (0, 0)),
        ],
        out_specs=pl.BlockSpec((BR, D_H), lambda i: (i, 0)),
        out_shape=jax.ShapeDtypeStruct((N, D_H), jnp.float32),
    )(degp, x, W1)

    p1 = _make_msg_kernel(D_H)(g1, src, dst, zeros128)

    g2 = pl.pallas_call(
        _tc2_body,
        grid=(N // BR,),
        in_specs=[
            _degp_spec(),
            pl.BlockSpec((NC, BR, D_H), lambda i: (0, i, 0)),
            pl.BlockSpec((BR, D_H), lambda i: (i, 0)),
            pl.BlockSpec((1, D_H), lambda i: (0, 0)),
            pl.BlockSpec((D_H, D_OUT), lambda i: (0, 0)),
        ],
        out_specs=pl.BlockSpec((BR, D_H), lambda i: (i, 0)),
        out_shape=jax.ShapeDtypeStruct((N, D_H), jnp.float32),
    )(degp, p1, g1, b1.reshape(1, D_H), W2)

    p2 = _make_msg_kernel(D_H)(g2, src, dst, zeros128)

    out = pl.pallas_call(
        _tc3_body,
        grid=(N // BR,),
        in_specs=[
            _degp_spec(),
            pl.BlockSpec((NC, BR, D_H), lambda i: (0, i, 0)),
            pl.BlockSpec((BR, D_H), lambda i: (i, 0)),
            pl.BlockSpec((1, D_OUT), lambda i: (0, 0)),
        ],
        out_specs=pl.BlockSpec((BR, D_OUT), lambda i: (i, 0)),
        out_shape=jax.ShapeDtypeStruct((N, D_OUT), jnp.float32),
    )(degp, p2, g2, b2.reshape(1, D_OUT))

    return out


# speedup vs baseline: 21.6508x; 1.3430x over previous
import functools

import jax
import jax.numpy as jnp
from jax import lax
from jax.experimental import pallas as pl
from jax.experimental.pallas import tpu as pltpu
from jax.experimental.pallas import tpu_sc as plsc

N = 10000
E = 320000
D_IN = 128
D_H = 128
D_OUT = 64

NC = 2
NS = 16
NW = NC * NS
CH = 96
K0 = 134
K1 = 76
KT = K0 + K1
TOTCH = NS * KT
TOTCH_ALLOC = TOTCH + (K0 - K1)
E_PAD = TOTCH_ALLOC * CH
KD = TOTCH // NW
N_ACC = 10016
RPT = N_ACC // NS
N_PAD = 10240
N_H = N_PAD
BR = 1024
BR2 = 1000

_mesh = plsc.VectorSubcoreMesh(core_axis_name="c", subcore_axis_name="s")


def _make_msg_kernel(d):
    @functools.partial(
        pl.kernel,
        out_type=jax.ShapeDtypeStruct((NC, N_ACC, d), jnp.float32),
        mesh=_mesh,
        scratch_types=[
            pltpu.VMEM((K0, CH), jnp.int32),
            pltpu.VMEM((K0, CH), jnp.int32),
            pltpu.VMEM((CH, d), jnp.float32),
            pltpu.VMEM((CH, d), jnp.float32),
            pltpu.VMEM_SHARED((N_ACC, d), jnp.float32),
            pltpu.SemaphoreType.DMA,
            pltpu.SemaphoreType.DMA,
        ],
        compiler_params=pltpu.CompilerParams(use_tc_tiling_on_sc=False),
    )
    def msg(g_hbm, src_hbm, dst_hbm, zero_hbm, out_hbm,
            sidx, didx, rows_a, rows_b, acc, sem_a, sem_b):
        c = lax.axis_index("c")
        s = lax.axis_index("s")
        r0 = s * RPT
        cb = jnp.where(c == 0, s * K0, NS * K0 + s * K1)
        kc = jnp.where(c == 0, K0, K1)
        pltpu.sync_copy(zero_hbm.at[pl.ds(r0, RPT)], acc.at[pl.ds(r0, RPT)])
        pltpu.sync_copy(src_hbm.at[pl.ds(cb, K0)], sidx)
        pltpu.sync_copy(dst_hbm.at[pl.ds(cb, K0)], didx)
        plsc.subcore_barrier()

        pltpu.async_copy(g_hbm.at[sidx.at[0]], rows_a, sem_a)

        def body(jj, carry):
            j0 = 2 * jj
            pltpu.make_async_copy(g_hbm.at[sidx.at[j0]], rows_a, sem_a).wait()
            pltpu.async_copy(g_hbm.at[sidx.at[j0 + 1]], rows_b, sem_b)
            pltpu.sync_copy(rows_a, acc.at[didx.at[j0]], add=True)
            jn = jnp.minimum(j0 + 2, kc - 1)
            pltpu.make_async_copy(g_hbm.at[sidx.at[j0 + 1]], rows_b, sem_b).wait()
            pltpu.async_copy(g_hbm.at[sidx.at[jn]], rows_a, sem_a)
            pltpu.sync_copy(rows_b, acc.at[didx.at[j0 + 1]], add=True)
            return carry

        lax.fori_loop(0, kc // 2, body, 0)
        pltpu.make_async_copy(g_hbm.at[sidx.at[kc - 1]], rows_a, sem_a).wait()
        plsc.subcore_barrier()
        pltpu.sync_copy(acc.at[pl.ds(r0, RPT)], out_hbm.at[c, pl.ds(r0, RPT)])

    return msg


@functools.partial(
    pl.kernel,
    out_type=jax.ShapeDtypeStruct((NW, N_H), jnp.float32),
    mesh=_mesh,
    scratch_types=[
        pltpu.VMEM((KD, CH), jnp.int32),
        pltpu.VMEM((N_H,), jnp.float32),
    ],
    compiler_params=pltpu.CompilerParams(
        use_tc_tiling_on_sc=False, needs_layout_passes=False),
)
def _deg_kernel(dst_hbm, out_hbm, didx, hist):
    c = lax.axis_index("c")
    s = lax.axis_index("s")
    wid = s * NC + c
    pltpu.sync_copy(dst_hbm.at[pl.ds(wid * KD, KD)], didx)

    def zbody(i, carry):
        hist[pl.ds(i * 16, 16)] = jnp.zeros((16,), jnp.float32)
        return carry

    lax.fori_loop(0, N_H // 16, zbody, 0)
    ones = jnp.full((16,), 1.0, jnp.float32)

    def body(j, carry):
        for l in range(CH // 16):
            iv = didx[j, pl.ds(l * 16, 16)]
            plsc.addupdate_scatter(hist, [iv], ones)
        return carry

    lax.fori_loop(0, KD, body, 0)
    pltpu.sync_copy(hist, out_hbm.at[wid])


def _tc1_body(degp_ref, x_ref, w_ref, g_ref, dinv_ref):
    deg = jnp.sum(degp_ref[...], axis=0) + 1.0
    dinv = lax.rsqrt(jnp.maximum(deg, 1.0))
    dinv_ref[...] = dinv[:, None]
    h = jnp.dot(x_ref[...], w_ref[...], preferred_element_type=jnp.float32)
    g_ref[...] = h * dinv[:, None]


def _tc2_body(dinv_ref, p_ref, g1_ref, b1_ref, w2_ref, g2_ref):
    dinv = dinv_ref[...]
    ssum = p_ref[0] + p_ref[1] + g1_ref[...]
    h = jnp.maximum(ssum * dinv + b1_ref[0], 0.0)
    h2 = jnp.dot(h, w2_ref[...], preferred_element_type=jnp.float32)
    g2_ref[...] = h2 * dinv


def _tc3_body(dinv_ref, p_ref, g2_ref, b2_ref, o_ref):
    dinv = dinv_ref[...]
    ssum = p_ref[0] + p_ref[1] + g2_ref[...]
    z = ssum * dinv + b2_ref[0]
    m = jnp.max(z, axis=1, keepdims=True)
    zz = z - m
    lse = jnp.log(jnp.sum(jnp.exp(zz), axis=1, keepdims=True))
    o_ref[...] = zz - lse


def kernel(x, edge_index, W1, b1, W2, b2):
    src = edge_index[0]
    dst = edge_index[1]
    pad = E_PAD - E
    src = jnp.concatenate([src, jnp.zeros((pad,), jnp.int32)]).reshape(TOTCH_ALLOC, CH)
    dst = jnp.concatenate([dst, jnp.full((pad,), N, jnp.int32)]).reshape(TOTCH_ALLOC, CH)

    xp = jnp.pad(x, ((0, N_PAD - N), (0, 0)))
    zeros128 = jnp.zeros((N_ACC, D_H), jnp.float32)

    degp = _deg_kernel(dst)

    g1, dinv = pl.pallas_call(
        _tc1_body,
        grid=(N_PAD // BR,),
        in_specs=[
            pl.BlockSpec((NW, BR), lambda i: (0, i)),
            pl.BlockSpec((BR, D_IN), lambda i: (i, 0)),
            pl.BlockSpec((D_IN, D_H), lambda i: (0, 0)),
        ],
        out_specs=[
            pl.BlockSpec((BR, D_H), lambda i: (i, 0)),
            pl.BlockSpec((BR, 1), lambda i: (i, 0)),
        ],
        out_shape=[
            jax.ShapeDtypeStruct((N_PAD, D_H), jnp.float32),
            jax.ShapeDtypeStruct((N_PAD, 1), jnp.float32),
        ],
    )(degp, xp, W1)

    p1 = _make_msg_kernel(D_H)(g1, src, dst, zeros128)

    g2 = pl.pallas_call(
        _tc2_body,
        grid=(N // BR2,),
        in_specs=[
            pl.BlockSpec((BR2, 1), lambda i: (i, 0)),
            pl.BlockSpec((NC, BR2, D_H), lambda i: (0, i, 0)),
            pl.BlockSpec((BR2, D_H), lambda i: (i, 0)),
            pl.BlockSpec((1, D_H), lambda i: (0, 0)),
            pl.BlockSpec((D_H, D_OUT), lambda i: (0, 0)),
        ],
        out_specs=pl.BlockSpec((BR2, D_OUT), lambda i: (i, 0)),
        out_shape=jax.ShapeDtypeStruct((N, D_OUT), jnp.float32),
    )(dinv, p1, g1, b1.reshape(1, D_H), W2)

    zeros64 = jnp.zeros((N_ACC, D_OUT), jnp.float32)
    p2 = _make_msg_kernel(D_OUT)(g2, src, dst, zeros64)

    out = pl.pallas_call(
        _tc3_body,
        grid=(N // BR2,),
        in_specs=[
            pl.BlockSpec((BR2, 1), lambda i: (i, 0)),
            pl.BlockSpec((NC, BR2, D_OUT), lambda i: (0, i, 0)),
            pl.BlockSpec((BR2, D_OUT), lambda i: (i, 0)),
            pl.BlockSpec((1, D_OUT), lambda i: (0, 0)),
        ],
        out_specs=pl.BlockSpec((BR2, D_OUT), lambda i: (i, 0)),
        out_shape=jax.ShapeDtypeStruct((N, D_OUT), jnp.float32),
    )(dinv, p2, g2, b2.reshape(1, D_OUT))

    return out


# speedup vs baseline: 28.7006x; 1.3256x over previous
import functools

import jax
import jax.numpy as jnp
from jax import lax
from jax.experimental import pallas as pl
from jax.experimental.pallas import tpu as pltpu
from jax.experimental.pallas import tpu_sc as plsc

N = 10000
E = 320000
D_IN = 128
D_H = 128
D_OUT = 64

NC = 2
NS = 16
NW = NC * NS
CH = 80
KPT = 250
TOTCH = NS * KPT
KD = TOTCH // NW
N_ACC = 10016
RPT = N_ACC // NS
N_PAD = 10240
N_H = N_PAD
BR = 1024
BR2 = 1000

_mesh = plsc.VectorSubcoreMesh(core_axis_name="c", subcore_axis_name="s")


def _make_msg_kernel(dh):
    @functools.partial(
        pl.kernel,
        out_type=jax.ShapeDtypeStruct((NC, N_ACC, dh), jnp.float32),
        mesh=_mesh,
        scratch_types=[
            pltpu.VMEM((KPT, CH), jnp.int32),
            pltpu.VMEM((KPT, CH), jnp.int32),
            pltpu.VMEM((CH, dh), jnp.float32),
            pltpu.VMEM((CH, dh), jnp.float32),
            pltpu.VMEM_SHARED((N_ACC, dh), jnp.float32),
            pltpu.VMEM_SHARED((N_ACC, dh), jnp.float32),
            pltpu.SemaphoreType.DMA,
            pltpu.SemaphoreType.DMA,
        ],
        compiler_params=pltpu.CompilerParams(use_tc_tiling_on_sc=False),
    )
    def msg(g_hbm, src_hbm, dst_hbm, zero_hbm, out_hbm,
            sidx, didx, rows_a, rows_b, table, acc, sem_a, sem_b):
        c = lax.axis_index("c")
        s = lax.axis_index("s")
        r0 = s * RPT
        pltpu.sync_copy(zero_hbm.at[pl.ds(r0, RPT)], acc.at[pl.ds(r0, RPT)])
        pltpu.sync_copy(g_hbm.at[c, pl.ds(r0, RPT)], table.at[pl.ds(r0, RPT)])
        pltpu.sync_copy(src_hbm.at[s], sidx)
        pltpu.sync_copy(dst_hbm.at[s], didx)
        plsc.subcore_barrier()

        pltpu.async_copy(table.at[sidx.at[0]], rows_a, sem_a)

        def body(jj, carry):
            j0 = 2 * jj
            pltpu.make_async_copy(table.at[sidx.at[j0]], rows_a, sem_a).wait()
            pltpu.async_copy(table.at[sidx.at[j0 + 1]], rows_b, sem_b)
            pltpu.sync_copy(rows_a, acc.at[didx.at[j0]], add=True)
            jn = jnp.minimum(j0 + 2, KPT - 1)
            pltpu.make_async_copy(table.at[sidx.at[j0 + 1]], rows_b, sem_b).wait()
            pltpu.async_copy(table.at[sidx.at[jn]], rows_a, sem_a)
            pltpu.sync_copy(rows_b, acc.at[didx.at[j0 + 1]], add=True)
            return carry

        lax.fori_loop(0, KPT // 2, body, 0)
        pltpu.make_async_copy(table.at[sidx.at[KPT - 1]], rows_a, sem_a).wait()
        plsc.subcore_barrier()
        pltpu.sync_copy(acc.at[pl.ds(r0, RPT)], out_hbm.at[c, pl.ds(r0, RPT)])

    return msg


@functools.partial(
    pl.kernel,
    out_type=jax.ShapeDtypeStruct((NW, N_H), jnp.float32),
    mesh=_mesh,
    scratch_types=[
        pltpu.VMEM((KD, CH), jnp.int32),
        pltpu.VMEM((N_H,), jnp.float32),
    ],
    compiler_params=pltpu.CompilerParams(
        use_tc_tiling_on_sc=False, needs_layout_passes=False),
)
def _deg_kernel(dst_hbm, out_hbm, didx, hist):
    c = lax.axis_index("c")
    s = lax.axis_index("s")
    wid = s * NC + c
    pltpu.sync_copy(dst_hbm.at[wid], didx)

    def zbody(i, carry):
        hist[pl.ds(i * 16, 16)] = jnp.zeros((16,), jnp.float32)
        return carry

    lax.fori_loop(0, N_H // 16, zbody, 0)
    ones = jnp.full((16,), 1.0, jnp.float32)

    def body(j, carry):
        for l in range(CH // 16):
            iv = didx[j, pl.ds(l * 16, 16)]
            plsc.addupdate_scatter(hist, [iv], ones)
        return carry

    lax.fori_loop(0, KD, body, 0)
    pltpu.sync_copy(hist, out_hbm.at[wid])


def _tc1_body(degp_ref, x_ref, w_ref, g_ref, dinv_ref):
    deg = jnp.sum(degp_ref[...], axis=0) + 1.0
    dinv = lax.rsqrt(jnp.maximum(deg, 1.0))
    dinv_ref[...] = dinv[:, None]
    h = jnp.dot(x_ref[...], w_ref[...], preferred_element_type=jnp.float32)
    g = h * dinv[:, None]
    g_ref[0] = g[:, :D_H // 2]
    g_ref[1] = g[:, D_H // 2:]


def _tc2_body(dinv_ref, p_ref, g1_ref, b1_ref, w2_ref, g2_ref):
    dinv = dinv_ref[...]
    ssum = jnp.concatenate([p_ref[0], p_ref[1]], axis=-1)
    ssum = ssum + jnp.concatenate([g1_ref[0], g1_ref[1]], axis=-1)
    h = jnp.maximum(ssum * dinv + b1_ref[0], 0.0)
    h2 = jnp.dot(h, w2_ref[...], preferred_element_type=jnp.float32)
    g2 = h2 * dinv
    g2_ref[0] = g2[:, :D_OUT // 2]
    g2_ref[1] = g2[:, D_OUT // 2:]


def _tc3_body(dinv_ref, p_ref, g2_ref, b2_ref, o_ref):
    dinv = dinv_ref[...]
    ssum = jnp.concatenate([p_ref[0], p_ref[1]], axis=-1)
    ssum = ssum + jnp.concatenate([g2_ref[0], g2_ref[1]], axis=-1)
    z = ssum * dinv + b2_ref[0]
    m = jnp.max(z, axis=1, keepdims=True)
    zz = z - m
    lse = jnp.log(jnp.sum(jnp.exp(zz), axis=1, keepdims=True))
    o_ref[...] = zz - lse


def kernel(x, edge_index, W1, b1, W2, b2):
    src = edge_index[0].reshape(NS, KPT, CH)
    dst = edge_index[1].reshape(NS, KPT, CH)
    dst_d = edge_index[1].reshape(NW, KD, CH)

    xp = jnp.pad(x, ((0, N_PAD - N), (0, 0)))
    zeros64 = jnp.zeros((N_ACC, D_H // 2), jnp.float32)
    zeros32 = jnp.zeros((N_ACC, D_OUT // 2), jnp.float32)

    degp = _deg_kernel(dst_d)

    g1, dinv = pl.pallas_call(
        _tc1_body,
        grid=(N_PAD // BR,),
        in_specs=[
            pl.BlockSpec((NW, BR), lambda i: (0, i)),
            pl.BlockSpec((BR, D_IN), lambda i: (i, 0)),
            pl.BlockSpec((D_IN, D_H), lambda i: (0, 0)),
        ],
        out_specs=[
            pl.BlockSpec((NC, BR, D_H // 2), lambda i: (0, i, 0)),
            pl.BlockSpec((BR, 1), lambda i: (i, 0)),
        ],
        out_shape=[
            jax.ShapeDtypeStruct((NC, N_PAD, D_H // 2), jnp.float32),
            jax.ShapeDtypeStruct((N_PAD, 1), jnp.float32),
        ],
    )(degp, xp, W1)

    p1 = _make_msg_kernel(D_H // 2)(g1, src, dst, zeros64)

    g2, = pl.pallas_call(
        _tc2_body,
        grid=(N // BR2,),
        in_specs=[
            pl.BlockSpec((BR2, 1), lambda i: (i, 0)),
            pl.BlockSpec((NC, BR2, D_H // 2), lambda i: (0, i, 0)),
            pl.BlockSpec((NC, BR2, D_H // 2), lambda i: (0, i, 0)),
            pl.BlockSpec((1, D_H), lambda i: (0, 0)),
            pl.BlockSpec((D_H, D_OUT), lambda i: (0, 0)),
        ],
        out_specs=[
            pl.BlockSpec((NC, BR2, D_OUT // 2), lambda i: (0, i, 0)),
        ],
        out_shape=[
            jax.ShapeDtypeStruct((NC, N, D_OUT // 2), jnp.float32),
        ],
    )(dinv, p1, g1, b1.reshape(1, D_H), W2)

    g2p = jnp.pad(g2, ((0, 0), (0, N_PAD - N), (0, 0)))
    p2 = _make_msg_kernel(D_OUT // 2)(g2p, src, dst, zeros32)

    out = pl.pallas_call(
        _tc3_body,
        grid=(N // BR2,),
        in_specs=[
            pl.BlockSpec((BR2, 1), lambda i: (i, 0)),
            pl.BlockSpec((NC, BR2, D_OUT // 2), lambda i: (0, i, 0)),
            pl.BlockSpec((NC, BR2, D_OUT // 2), lambda i: (0, i, 0)),
            pl.BlockSpec((1, D_OUT), lambda i: (0, 0)),
        ],
        out_specs=pl.BlockSpec((BR2, D_OUT), lambda i: (i, 0)),
        out_shape=jax.ShapeDtypeStruct((N, D_OUT), jnp.float32),
    )(dinv, p2, g2, b2.reshape(1, D_OUT))

    return out
